# baseline (speedup 1.0000x reference)
import jax
import jax.numpy as jnp
from jax import lax
from jax.experimental import pallas as pl
from jax.experimental.pallas import tpu as pltpu
from jax.experimental.pallas import tpu_sc as plsc

B, S1, S2 = 4096, 200, 50
V1 = 1000000
NC, NS, L = 2, 16, 16
NW = NC * NS
BPW = B // NW
NG = BPW // L
SCH = 40
NCHUNK = S1 // SCH
NSLICE = SCH * L // 128

RCH = 4000
RNCH = V1 // RCH
RBLK = RCH // L

_PARAMS = pltpu.CompilerParams(use_tc_tiling_on_sc=False,
                               needs_layout_passes=False)


def _repack_body(f1t_h, f2t_h, f3t_h, o1_h, o2_h, o3_h,
                 colbuf, rows16, sem):
    cid = lax.axis_index("c")
    sid = lax.axis_index("s")
    wid = sid * NC + cid
    iota = lax.iota(jnp.int32, L)

    def splat(v):
        return jnp.full((L,), v, jnp.int32)

    for src_h, out_h, ncols in ((f1t_h, o1_h, 12), (f2t_h, o2_h, 6),
                                (f3t_h, o3_h, 8)):
        def chunk_body(i, carry, src_h=src_h, out_h=out_h, ncols=ncols):
            q = wid + i * NW

            @pl.when(q < RNCH)
            def _():
                r0 = q * RCH
                descs = [pltpu.async_copy(
                    src_h.at[c, pl.ds(r0, RCH)], colbuf.at[c], sem)
                    for c in range(ncols)]
                for d in descs:
                    d.wait()

                def blk(k, carry2):
                    rv = k * L + iota
                    for c in range(ncols):
                        v = colbuf[c, pl.ds(k * L, L)]
                        plsc.store_scatter(rows16, [rv, splat(c)], v)
                    return carry2

                lax.fori_loop(0, RBLK, blk, 0)
                pltpu.async_copy(rows16, out_h.at[pl.ds(r0, RCH)], sem).wait()
            return carry

        lax.fori_loop(0, (RNCH + NW - 1) // NW, chunk_body, 0)


def _body(msgs_h, mmask_h, vb_h, vmask_h, sm_h,
          f1_h, f2_h, f3_h, es_h, ev_h, w1_h, b1_h, w2_h, b2_h,
          out_h,
          msgs_v, idx_bufs, rows1, rows2, rows3, mmask_v,
          venus_v, vmask_v, sm_v, ev_v, es_v, w1_v, b1_v, w2_v, b2_v,
          feat, out_v, sem):
    cid = lax.axis_index("c")
    sid = lax.axis_index("s")
    wid = sid * NC + cid
    b0w = wid * BPW

    pltpu.sync_copy(ev_h, ev_v)
    pltpu.sync_copy(es_h, es_v)
    pltpu.sync_copy(w1_h, w1_v)
    pltpu.sync_copy(b1_h, b1_v)
    pltpu.sync_copy(w2_h, w2_v)
    pltpu.sync_copy(b2_h, b2_v)
    pltpu.sync_copy(sm_h.at[pl.ds(b0w, BPW)], sm_v)

    iota = lax.iota(jnp.int32, L)
    zero = jnp.zeros((L,), jnp.float32)

    def splat(v):
        return jnp.full((L,), v, jnp.int32)

    b1vec = b1_v[...]
    b2vec = b2_v[...]
    b1s = [b1vec[j] for j in range(10)]
    b2s = [b2vec[o] for o in range(4)]
    w2rows = [w2_v[pl.ds(16 * j, L)] for j in range(10)]
    w2s = [[w2rows[j][o] for o in range(4)] for j in range(10)]

    def group_body(g, carry):
        b0 = b0w + g * L

        pltpu.sync_copy(msgs_h.at[pl.ds(b0, L)], msgs_v)
        pltpu.sync_copy(mmask_h.at[pl.ds(b0, L)], mmask_v)
        pltpu.sync_copy(vb_h.at[pl.ds(b0, L)], venus_v)
        pltpu.sync_copy(vmask_h.at[pl.ds(b0, L)], vmask_v)

        accs = (zero,) * 26
        for ci in range(NCHUNK):
            for t in range(SCH):
                s_glob = ci * SCH + t
                dst = t // 8
                off = (t % 8) * L
                for j in range(3):
                    v = plsc.load_gather(msgs_v, [iota, splat(3 * s_glob + j)])
                    idx_bufs[j][dst][pl.ds(off, L)] = v
            descs = []
            for k in range(NSLICE):
                descs.append(pltpu.async_copy(
                    f1_h.at[idx_bufs[0][k]], rows1.at[pl.ds(k * 128, 128)], sem))
                descs.append(pltpu.async_copy(
                    f2_h.at[idx_bufs[1][k]], rows2.at[pl.ds(k * 128, 128)], sem))
                descs.append(pltpu.async_copy(
                    f3_h.at[idx_bufs[2][k]], rows3.at[pl.ds(k * 128, 128)], sem))
            for d in descs:
                d.wait()

            def s_body(s, a):
                m = plsc.load_gather(mmask_v, [iota, splat(ci * SCH) + s])
                rvec = s * L + iota
                new = []
                for c in range(12):
                    new.append(a[c] + m * plsc.load_gather(rows1, [rvec, splat(c)]))
                for c in range(6):
                    new.append(a[12 + c] + m * plsc.load_gather(rows2, [rvec, splat(c)]))
                for c in range(8):
                    new.append(a[18 + c] + m * plsc.load_gather(rows3, [rvec, splat(c)]))
                return tuple(new)

            accs = lax.fori_loop(0, SCH, s_body, accs)
        for c in range(26):
            feat[c] = accs[c]

        smv = plsc.load_gather(sm_v, [g * L + iota])
        for c in range(8):
            feat[26 + c] = plsc.load_gather(es_v, [smv * 8 + splat(c)])

        def vs_body(s, a):
            mv = plsc.load_gather(vmask_v, [iota, jnp.full((L,), s, jnp.int32)])
            new = list(a)
            for j in range(3):
                tix = plsc.load_gather(venus_v, [iota, 3 * s + splat(j)])
                for c in range(4):
                    val = plsc.load_gather(ev_v, [tix * 4 + splat(c)])
                    new[j * 4 + c] = new[j * 4 + c] + mv * val
            return tuple(new)

        vaccs = lax.fori_loop(0, S2, vs_body, (zero,) * 12)
        for c in range(12):
            feat[34 + c] = vaccs[c]

        def k_body(k, acc):
            fk = feat[k]
            w1row = w1_v[pl.ds(k * L, L)]
            return tuple(acc[j] + fk * w1row[j] for j in range(10))

        acc10 = lax.fori_loop(
            0, 46, k_body,
            tuple(jnp.full((L,), b1s[j], jnp.float32) for j in range(10)))
        h = [jnp.maximum(a, 0.0) for a in acc10]
        rvec = (g * L + iota) * 4
        for o in range(4):
            v = jnp.full((L,), b2s[o], jnp.float32)
            for j in range(10):
                v = v + h[j] * w2s[j][o]
            plsc.store_scatter(out_v, [rvec + splat(o)], v)
        return carry

    lax.fori_loop(0, NG, group_body, 0)
    pltpu.sync_copy(out_v, out_h.at[pl.ds(b0w * 4, BPW * 4)])


@jax.jit
def _run(msgs, msg_mask, venus_batch, venus_mask, server_model,
         emb_f1, emb_f2, emb_f3, emb_server, emb_venus, W1, b1, W2, b2):
    mesh = plsc.VectorSubcoreMesh(
        core_axis_name="c", subcore_axis_name="s",
        num_cores=NC, num_subcores=NS)

    repack = pl.kernel(
        _repack_body,
        out_type=(jax.ShapeDtypeStruct((V1, 16), jnp.float32),
                  jax.ShapeDtypeStruct((V1, 16), jnp.float32),
                  jax.ShapeDtypeStruct((V1, 16), jnp.float32)),
        mesh=mesh,
        scratch_types=[
            pltpu.VMEM((12, RCH), jnp.float32),
            pltpu.VMEM((RCH, 16), jnp.float32),
            pltpu.SemaphoreType.DMA,
        ],
        compiler_params=_PARAMS,
    )
    t1, t2, t3 = repack(emb_f1.T, emb_f2.T, emb_f3.T)

    scratch = [
        pltpu.VMEM((L, S1 * 3), jnp.int32),
        [[pltpu.VMEM((128,), jnp.int32) for _ in range(NSLICE)]
         for _ in range(3)],
        pltpu.VMEM((SCH * L, 16), jnp.float32),
        pltpu.VMEM((SCH * L, 16), jnp.float32),
        pltpu.VMEM((SCH * L, 16), jnp.float32),
        pltpu.VMEM((L, S1), jnp.float32),
        pltpu.VMEM((L, S2 * 3), jnp.int32),
        pltpu.VMEM((L, S2), jnp.float32),
        pltpu.VMEM((BPW,), jnp.int32),
        pltpu.VMEM((4000,), jnp.float32),
        pltpu.VMEM((704,), jnp.float32),
        pltpu.VMEM((46 * L,), jnp.float32),
        pltpu.VMEM((L,), jnp.float32),
        pltpu.VMEM((10 * L,), jnp.float32),
        pltpu.VMEM((L,), jnp.float32),
        pltpu.VMEM((46, L), jnp.float32),
        pltpu.VMEM((BPW * 4,), jnp.float32),
        pltpu.SemaphoreType.DMA,
    ]
    run = pl.kernel(
        _body,
        out_type=jax.ShapeDtypeStruct((B * 4,), jnp.float32),
        mesh=mesh,
        scratch_types=scratch,
        compiler_params=_PARAMS,
    )
    w1p = jnp.zeros((46, L), jnp.float32).at[:, :10].set(W1).reshape(-1)
    b1p = jnp.zeros((L,), jnp.float32).at[:10].set(b1)
    w2p = jnp.zeros((10, L), jnp.float32).at[:, :4].set(W2).reshape(-1)
    b2p = jnp.zeros((L,), jnp.float32).at[:4].set(b2)
    out = run(msgs.reshape(B, S1 * 3), msg_mask,
              venus_batch.reshape(B, S2 * 3), venus_mask, server_model,
              t1, t2, t3,
              emb_server.reshape(-1), emb_venus.reshape(-1),
              w1p, b1p, w2p, b2p)
    return out.reshape(B, 4)


def kernel(msgs, msg_mask, venus_batch, venus_mask, server_model, crash_dump,
           emb_f1, emb_f2, emb_f3, emb_server, emb_venus, W1, b1, W2, b2):
    del crash_dump
    return _run(msgs, msg_mask, venus_batch, venus_mask, server_model,
                emb_f1, emb_f2, emb_f3, emb_server, emb_venus, W1, b1, W2, b2)


# speedup vs baseline: 1.5291x; 1.5291x over previous
import jax
import jax.numpy as jnp
from jax import lax
from jax.experimental import pallas as pl
from jax.experimental.pallas import tpu as pltpu
from jax.experimental.pallas import tpu_sc as plsc

B, S1, S2 = 4096, 200, 50
V1 = 1000000
NC, NS, L = 2, 16, 16
NW = NC * NS
BPW = B // NW
NG = BPW // L
SCH = 40
NCHUNK = S1 // SCH
NSLICE = SCH * L // 128

_PARAMS = pltpu.CompilerParams(use_tc_tiling_on_sc=False,
                               needs_layout_passes=False)


def _body(msgs_h, mmask_h, vb_h, vmask_h, sm_h,
          f1_h, f2_h, f3_h, es_h, ev_h, w1_h, b1_h, w2_h, b2_h,
          out_h,
          msgs_v, idx_bufs, rows1, rows2, rows3, cof1, cof2, mmask_v,
          venus_v, vmask_v, sm_v, ev_v, es_v, w1_v, b1_v, w2_v, b2_v,
          feat, out_v, sem):
    cid = lax.axis_index("c")
    sid = lax.axis_index("s")
    wid = sid * NC + cid
    b0w = wid * BPW

    pltpu.sync_copy(ev_h, ev_v)
    pltpu.sync_copy(es_h, es_v)
    pltpu.sync_copy(w1_h, w1_v)
    pltpu.sync_copy(b1_h, b1_v)
    pltpu.sync_copy(w2_h, w2_v)
    pltpu.sync_copy(b2_h, b2_v)
    pltpu.sync_copy(sm_h.at[pl.ds(b0w, BPW)], sm_v)

    iota = lax.iota(jnp.int32, L)
    zero = jnp.zeros((L,), jnp.float32)

    def splat(v):
        return jnp.full((L,), v, jnp.int32)

    b1vec = b1_v[...]
    b2vec = b2_v[...]
    b1s = [b1vec[j] for j in range(10)]
    b2s = [b2vec[o] for o in range(4)]
    w2rows = [w2_v[pl.ds(16 * j, L)] for j in range(10)]
    w2s = [[w2rows[j][o] for o in range(4)] for j in range(10)]

    def group_body(g, carry):
        b0 = b0w + g * L

        pltpu.sync_copy(msgs_h.at[pl.ds(b0, L)], msgs_v)
        pltpu.sync_copy(mmask_h.at[pl.ds(b0, L)], mmask_v)
        pltpu.sync_copy(vb_h.at[pl.ds(b0, L)], venus_v)
        pltpu.sync_copy(vmask_h.at[pl.ds(b0, L)], vmask_v)

        accs = (zero,) * 26
        for ci in range(NCHUNK):
            for t in range(SCH):
                s_glob = ci * SCH + t
                dst = t // 8
                off = (t % 8) * L
                v1 = plsc.load_gather(msgs_v, [iota, splat(3 * s_glob)])
                idx_bufs[0][dst][pl.ds(off, L)] = v1 >> 1
                cof1[pl.ds(t * L, L)] = (v1 & 1) * 12
                v2 = plsc.load_gather(msgs_v, [iota, splat(3 * s_glob + 1)])
                idx_bufs[1][dst][pl.ds(off, L)] = v2 >> 2
                cof2[pl.ds(t * L, L)] = (v2 & 3) * 6
                v3 = plsc.load_gather(msgs_v, [iota, splat(3 * s_glob + 2)])
                idx_bufs[2][dst][pl.ds(off, L)] = v3
            descs = []
            for k in range(NSLICE):
                descs.append(pltpu.async_copy(
                    f1_h.at[idx_bufs[0][k]], rows1.at[pl.ds(k * 128, 128)], sem))
                descs.append(pltpu.async_copy(
                    f2_h.at[idx_bufs[1][k]], rows2.at[pl.ds(k * 128, 128)], sem))
                descs.append(pltpu.async_copy(
                    f3_h.at[idx_bufs[2][k]], rows3.at[pl.ds(k * 128, 128)], sem))
            for d in descs:
                d.wait()

            def s_body(s, a):
                m = plsc.load_gather(mmask_v, [iota, splat(ci * SCH) + s])
                rvec = s * L + iota
                c1 = plsc.load_gather(cof1, [s * L + iota])
                c2 = plsc.load_gather(cof2, [s * L + iota])
                new = []
                for c in range(12):
                    new.append(a[c] + m * plsc.load_gather(rows1, [rvec, c1 + splat(c)]))
                for c in range(6):
                    new.append(a[12 + c] + m * plsc.load_gather(rows2, [rvec, c2 + splat(c)]))
                for c in range(8):
                    new.append(a[18 + c] + m * plsc.load_gather(rows3, [rvec, splat(c)]))
                return tuple(new)

            accs = lax.fori_loop(0, SCH, s_body, accs)
        for c in range(26):
            feat[c] = accs[c]

        smv = plsc.load_gather(sm_v, [g * L + iota])
        for c in range(8):
            feat[26 + c] = plsc.load_gather(es_v, [smv * 8 + splat(c)])

        def vs_body(s, a):
            mv = plsc.load_gather(vmask_v, [iota, jnp.full((L,), s, jnp.int32)])
            new = list(a)
            for j in range(3):
                tix = plsc.load_gather(venus_v, [iota, 3 * s + splat(j)])
                for c in range(4):
                    val = plsc.load_gather(ev_v, [tix * 4 + splat(c)])
                    new[j * 4 + c] = new[j * 4 + c] + mv * val
            return tuple(new)

        vaccs = lax.fori_loop(0, S2, vs_body, (zero,) * 12)
        for c in range(12):
            feat[34 + c] = vaccs[c]

        def k_body(k, acc):
            fk = feat[k]
            w1row = w1_v[pl.ds(k * L, L)]
            return tuple(acc[j] + fk * w1row[j] for j in range(10))

        acc10 = lax.fori_loop(
            0, 46, k_body,
            tuple(jnp.full((L,), b1s[j], jnp.float32) for j in range(10)))
        h = [jnp.maximum(a, 0.0) for a in acc10]
        rvec = (g * L + iota) * 4
        for o in range(4):
            v = jnp.full((L,), b2s[o], jnp.float32)
            for j in range(10):
                v = v + h[j] * w2s[j][o]
            plsc.store_scatter(out_v, [rvec + splat(o)], v)
        return carry

    lax.fori_loop(0, NG, group_body, 0)
    pltpu.sync_copy(out_v, out_h.at[pl.ds(b0w * 4, BPW * 4)])


@jax.jit
def _run(msgs, msg_mask, venus_batch, venus_mask, server_model,
         emb_f1, emb_f2, emb_f3, emb_server, emb_venus, W1, b1, W2, b2):
    mesh = plsc.VectorSubcoreMesh(
        core_axis_name="c", subcore_axis_name="s",
        num_cores=NC, num_subcores=NS)

    scratch = [
        pltpu.VMEM((L, S1 * 3), jnp.int32),
        [[pltpu.VMEM((128,), jnp.int32) for _ in range(NSLICE)]
         for _ in range(3)],
        pltpu.VMEM((SCH * L, 24), jnp.float32),
        pltpu.VMEM((SCH * L, 24), jnp.float32),
        pltpu.VMEM((SCH * L, 8), jnp.float32),
        pltpu.VMEM((SCH * L,), jnp.int32),
        pltpu.VMEM((SCH * L,), jnp.int32),
        pltpu.VMEM((L, S1), jnp.float32),
        pltpu.VMEM((L, S2 * 3), jnp.int32),
        pltpu.VMEM((L, S2), jnp.float32),
        pltpu.VMEM((BPW,), jnp.int32),
        pltpu.VMEM((4000,), jnp.float32),
        pltpu.VMEM((704,), jnp.float32),
        pltpu.VMEM((46 * L,), jnp.float32),
        pltpu.VMEM((L,), jnp.float32),
        pltpu.VMEM((10 * L,), jnp.float32),
        pltpu.VMEM((L,), jnp.float32),
        pltpu.VMEM((46, L), jnp.float32),
        pltpu.VMEM((BPW * 4,), jnp.float32),
        pltpu.SemaphoreType.DMA,
    ]
    run = pl.kernel(
        _body,
        out_type=jax.ShapeDtypeStruct((B * 4,), jnp.float32),
        mesh=mesh,
        scratch_types=scratch,
        compiler_params=_PARAMS,
    )
    w1p = jnp.zeros((46, L), jnp.float32).at[:, :10].set(W1).reshape(-1)
    b1p = jnp.zeros((L,), jnp.float32).at[:10].set(b1)
    w2p = jnp.zeros((10, L), jnp.float32).at[:, :4].set(W2).reshape(-1)
    b2p = jnp.zeros((L,), jnp.float32).at[:4].set(b2)
    out = run(msgs.reshape(B, S1 * 3), msg_mask,
              venus_batch.reshape(B, S2 * 3), venus_mask, server_model,
              emb_f1.reshape(V1 // 2, 24), emb_f2.reshape(V1 // 4, 24),
              emb_f3,
              emb_server.reshape(-1), emb_venus.reshape(-1),
              w1p, b1p, w2p, b2p)
    return out.reshape(B, 4)


def kernel(msgs, msg_mask, venus_batch, venus_mask, server_model, crash_dump,
           emb_f1, emb_f2, emb_f3, emb_server, emb_venus, W1, b1, W2, b2):
    del crash_dump
    return _run(msgs, msg_mask, venus_batch, venus_mask, server_model,
                emb_f1, emb_f2, emb_f3, emb_server, emb_venus, W1, b1, W2, b2)


# speedup vs baseline: 1.6065x; 1.0506x over previous
import jax
import jax.numpy as jnp
from jax import lax
from jax.experimental import pallas as pl
from jax.experimental.pallas import tpu as pltpu
from jax.experimental.pallas import tpu_sc as plsc

B, S1, S2 = 4096, 200, 50
V1 = 1000000
NC, NS, L = 2, 16, 16
NW = NC * NS
BPW = B // NW
NG = BPW // L
SCH = 40
NCHUNK = S1 // SCH
NSLICE = SCH * L // 128

_PARAMS = pltpu.CompilerParams(use_tc_tiling_on_sc=False,
                               needs_layout_passes=False)


def _body(msgs_h, mmask_h, vb_h, vmask_h, sm_h,
          f1_h, f2_h, f3_h, es_h, ev_h, w1_h, b1_h, w2_h, b2_h,
          out_h,
          msgs_v, idx_bufs, rows1, rows2, rows3, cof1, cof2, mmask_v,
          venus_v, vmask_v, sm_v, ev_v, es_v, w1_v, b1_v, w2_v, b2_v,
          feat, out_v, sem):
    cid = lax.axis_index("c")
    sid = lax.axis_index("s")
    wid = sid * NC + cid
    b0w = wid * BPW

    pltpu.sync_copy(ev_h, ev_v)
    pltpu.sync_copy(es_h, es_v)
    pltpu.sync_copy(w1_h, w1_v)
    pltpu.sync_copy(b1_h, b1_v)
    pltpu.sync_copy(w2_h, w2_v)
    pltpu.sync_copy(b2_h, b2_v)
    pltpu.sync_copy(sm_h.at[pl.ds(b0w, BPW)], sm_v)

    iota = lax.iota(jnp.int32, L)
    zero = jnp.zeros((L,), jnp.float32)

    def splat(v):
        return jnp.full((L,), v, jnp.int32)

    b1vec = b1_v[...]
    b2vec = b2_v[...]
    b1s = [b1vec[j] for j in range(10)]
    b2s = [b2vec[o] for o in range(4)]
    w2rows = [w2_v[pl.ds(16 * j, L)] for j in range(10)]
    w2s = [[w2rows[j][o] for o in range(4)] for j in range(10)]

    def group_body(g, carry):
        b0 = b0w + g * L

        pltpu.sync_copy(msgs_h.at[pl.ds(b0, L)], msgs_v)
        pltpu.sync_copy(mmask_h.at[pl.ds(b0, L)], mmask_v)
        pltpu.sync_copy(vb_h.at[pl.ds(b0, L)], venus_v)
        pltpu.sync_copy(vmask_h.at[pl.ds(b0, L)], vmask_v)

        def extract(ci, pb):
            for t in range(SCH):
                s_glob = ci * SCH + t
                dst = t // 8
                off = (t % 8) * L
                v1 = plsc.load_gather(msgs_v, [iota, splat(3 * s_glob)])
                idx_bufs[pb][0][dst][pl.ds(off, L)] = v1 >> 1
                cof1[pb][pl.ds(t * L, L)] = (v1 & 1) * 12
                v2 = plsc.load_gather(msgs_v, [iota, splat(3 * s_glob + 1)])
                idx_bufs[pb][1][dst][pl.ds(off, L)] = v2 >> 2
                cof2[pb][pl.ds(t * L, L)] = (v2 & 3) * 6
                v3 = plsc.load_gather(msgs_v, [iota, splat(3 * s_glob + 2)])
                idx_bufs[pb][2][dst][pl.ds(off, L)] = v3

        def fire(pb):
            descs = []
            for k in range(NSLICE):
                descs.append(pltpu.async_copy(
                    f1_h.at[idx_bufs[pb][0][k]],
                    rows1[pb].at[pl.ds(k * 128, 128)], sem[pb]))
                descs.append(pltpu.async_copy(
                    f2_h.at[idx_bufs[pb][1][k]],
                    rows2[pb].at[pl.ds(k * 128, 128)], sem[pb]))
                descs.append(pltpu.async_copy(
                    f3_h.at[idx_bufs[pb][2][k]],
                    rows3[pb].at[pl.ds(k * 128, 128)], sem[pb]))
            return descs

        def accumulate(ci, pb, accs):
            def s_body(s, a):
                m = plsc.load_gather(mmask_v, [iota, splat(ci * SCH) + s])
                rvec = s * L + iota
                c1 = plsc.load_gather(cof1[pb], [s * L + iota])
                c2 = plsc.load_gather(cof2[pb], [s * L + iota])
                new = []
                for c in range(12):
                    new.append(a[c] + m * plsc.load_gather(
                        rows1[pb], [rvec, c1 + splat(c)]))
                for c in range(6):
                    new.append(a[12 + c] + m * plsc.load_gather(
                        rows2[pb], [rvec, c2 + splat(c)]))
                for c in range(8):
                    new.append(a[18 + c] + m * plsc.load_gather(
                        rows3[pb], [rvec, splat(c)]))
                return tuple(new)

            return lax.fori_loop(0, SCH, s_body, accs)

        accs = (zero,) * 26
        extract(0, 0)
        inflight = {0: fire(0)}
        for ci in range(NCHUNK):
            pb = ci % 2
            if ci + 1 < NCHUNK:
                npb = (ci + 1) % 2
                extract(ci + 1, npb)
                inflight[npb] = fire(npb)
            for d in inflight[pb]:
                d.wait()
            accs = accumulate(ci, pb, accs)
        for c in range(26):
            feat[c] = accs[c]

        smv = plsc.load_gather(sm_v, [g * L + iota])
        for c in range(8):
            feat[26 + c] = plsc.load_gather(es_v, [smv * 8 + splat(c)])

        def vs_body(s, a):
            mv = plsc.load_gather(vmask_v, [iota, jnp.full((L,), s, jnp.int32)])
            new = list(a)
            for j in range(3):
                tix = plsc.load_gather(venus_v, [iota, 3 * s + splat(j)])
                for c in range(4):
                    val = plsc.load_gather(ev_v, [tix * 4 + splat(c)])
                    new[j * 4 + c] = new[j * 4 + c] + mv * val
            return tuple(new)

        vaccs = lax.fori_loop(0, S2, vs_body, (zero,) * 12)
        for c in range(12):
            feat[34 + c] = vaccs[c]

        def k_body(k, acc):
            fk = feat[k]
            w1row = w1_v[pl.ds(k * L, L)]
            return tuple(acc[j] + fk * w1row[j] for j in range(10))

        acc10 = lax.fori_loop(
            0, 46, k_body,
            tuple(jnp.full((L,), b1s[j], jnp.float32) for j in range(10)))
        h = [jnp.maximum(a, 0.0) for a in acc10]
        rvec = (g * L + iota) * 4
        for o in range(4):
            v = jnp.full((L,), b2s[o], jnp.float32)
            for j in range(10):
                v = v + h[j] * w2s[j][o]
            plsc.store_scatter(out_v, [rvec + splat(o)], v)
        return carry

    lax.fori_loop(0, NG, group_body, 0)
    pltpu.sync_copy(out_v, out_h.at[pl.ds(b0w * 4, BPW * 4)])


@jax.jit
def _run(msgs, msg_mask, venus_batch, venus_mask, server_model,
         emb_f1, emb_f2, emb_f3, emb_server, emb_venus, W1, b1, W2, b2):
    mesh = plsc.VectorSubcoreMesh(
        core_axis_name="c", subcore_axis_name="s",
        num_cores=NC, num_subcores=NS)

    scratch = [
        pltpu.VMEM((L, S1 * 3), jnp.int32),
        [[[pltpu.VMEM((128,), jnp.int32) for _ in range(NSLICE)]
          for _ in range(3)] for _ in range(2)],
        [pltpu.VMEM((SCH * L, 24), jnp.float32) for _ in range(2)],
        [pltpu.VMEM((SCH * L, 24), jnp.float32) for _ in range(2)],
        [pltpu.VMEM((SCH * L, 8), jnp.float32) for _ in range(2)],
        [pltpu.VMEM((SCH * L,), jnp.int32) for _ in range(2)],
        [pltpu.VMEM((SCH * L,), jnp.int32) for _ in range(2)],
        pltpu.VMEM((L, S1), jnp.float32),
        pltpu.VMEM((L, S2 * 3), jnp.int32),
        pltpu.VMEM((L, S2), jnp.float32),
        pltpu.VMEM((BPW,), jnp.int32),
        pltpu.VMEM((4000,), jnp.float32),
        pltpu.VMEM((704,), jnp.float32),
        pltpu.VMEM((46 * L,), jnp.float32),
        pltpu.VMEM((L,), jnp.float32),
        pltpu.VMEM((10 * L,), jnp.float32),
        pltpu.VMEM((L,), jnp.float32),
        pltpu.VMEM((46, L), jnp.float32),
        pltpu.VMEM((BPW * 4,), jnp.float32),
        [pltpu.SemaphoreType.DMA for _ in range(2)],
    ]
    run = pl.kernel(
        _body,
        out_type=jax.ShapeDtypeStruct((B * 4,), jnp.float32),
        mesh=mesh,
        scratch_types=scratch,
        compiler_params=_PARAMS,
    )
    w1p = jnp.zeros((46, L), jnp.float32).at[:, :10].set(W1).reshape(-1)
    b1p = jnp.zeros((L,), jnp.float32).at[:10].set(b1)
    w2p = jnp.zeros((10, L), jnp.float32).at[:, :4].set(W2).reshape(-1)
    b2p = jnp.zeros((L,), jnp.float32).at[:4].set(b2)
    out = run(msgs.reshape(B, S1 * 3), msg_mask,
              venus_batch.reshape(B, S2 * 3), venus_mask, server_model,
              emb_f1.reshape(V1 // 2, 24), emb_f2.reshape(V1 // 4, 24),
              emb_f3,
              emb_server.reshape(-1), emb_venus.reshape(-1),
              w1p, b1p, w2p, b2p)
    return out.reshape(B, 4)


def kernel(msgs, msg_mask, venus_batch, venus_mask, server_model, crash_dump,
           emb_f1, emb_f2, emb_f3, emb_server, emb_venus, W1, b1, W2, b2):
    del crash_dump
    return _run(msgs, msg_mask, venus_batch, venus_mask, server_model,
                emb_f1, emb_f2, emb_f3, emb_server, emb_venus, W1, b1, W2, b2)
